# baseline (speedup 1.0000x reference)
import functools

import jax
import jax.numpy as jnp
from jax.experimental import pallas as pl
from jax.experimental.pallas import tpu as pltpu

K = 32
Q_BLK = 128
R = 5
LSUB = 128
BIGN = 1 << 30


def _ce(a, an, b, bn):
    sw = b < a
    return (jnp.where(sw, b, a), jnp.where(sw, bn, an),
            jnp.where(sw, a, b), jnp.where(sw, an, bn))


def _knn_kernel(q_ref, pt_ref, o_ref, d_ref):
    q = q_ref[0]
    pt = pt_ref[0]
    n = pt.shape[1]
    s_rows = n // LSUB
    qn = jnp.sum(q * q, axis=1, keepdims=True)
    pn = jnp.sum(pt * pt, axis=0, keepdims=True)
    d = jax.lax.dot_general(
        q, pt, (((1,), (0,)), ((), ())),
        preferred_element_type=jnp.float32)
    d = (-2.0 * d + qn) + pn
    d_ref[...] = d.reshape(Q_BLK, s_rows, LSUB)

    inf = jnp.float32(jnp.inf)
    lane = jax.lax.broadcasted_iota(jnp.int32, (Q_BLK, LSUB), 1)
    k_lane = jax.lax.broadcasted_iota(jnp.int32, (Q_BLK, K), 1)

    def ins(s, carry):
        (v0, v1, v2, v3, v4, n0, n1, n2, n3, n4) = carry
        c = d_ref[:, s, :]
        cn = lane + s * LSUB
        lt = c < v4
        v4n = jnp.where(lt, c, v4)
        n4n = jnp.where(lt, cn, n4)
        v3, n3, v4, n4 = _ce(v3, n3, v4n, n4n)
        v2, n2, v3, n3 = _ce(v2, n2, v3, n3)
        v1, n1, v2, n2 = _ce(v1, n1, v2, n2)
        v0, n0, v1, n1 = _ce(v0, n0, v1, n1)
        return (v0, v1, v2, v3, v4, n0, n1, n2, n3, n4)

    finf = jnp.full((Q_BLK, LSUB), inf, jnp.float32)
    fbig = jnp.full((Q_BLK, LSUB), BIGN, jnp.int32)
    carry = (finf, finf, finf, finf, finf, fbig, fbig, fbig, fbig, fbig)
    (v0, v1, v2, v3, v4, n0, n1, n2, n3, n4) = jax.lax.fori_loop(
        0, s_rows, ins, carry)

    out = jnp.zeros((Q_BLK, K), jnp.int32)
    for k in range(K):
        mv = jnp.min(v0, axis=1, keepdims=True)
        sel = v0 == mv
        mn = jnp.min(jnp.where(sel, n0, BIGN), axis=1, keepdims=True)
        out = jnp.where(k_lane == k, mn, out)
        win = sel & (n0 == mn)
        v0 = jnp.where(win, v1, v0)
        n0 = jnp.where(win, n1, n0)
        v1 = jnp.where(win, v2, v1)
        n1 = jnp.where(win, n2, n1)
        v2 = jnp.where(win, v3, v2)
        n2 = jnp.where(win, n3, n2)
        v3 = jnp.where(win, v4, v3)
        n3 = jnp.where(win, n4, n3)
        v4 = jnp.where(win, inf, v4)
        n4 = jnp.where(win, BIGN, n4)
    o_ref[0, :, :] = out


def kernel(xyz, new_xyz):
    b, n, _ = xyz.shape
    m = new_xyz.shape[1]
    xyz_t = jnp.swapaxes(xyz, 1, 2)
    grid = (b, m // Q_BLK)
    return pl.pallas_call(
        _knn_kernel,
        grid=grid,
        in_specs=[
            pl.BlockSpec((1, Q_BLK, 3), lambda bi, qi: (bi, qi, 0)),
            pl.BlockSpec((1, 3, n), lambda bi, qi: (bi, 0, 0)),
        ],
        out_specs=pl.BlockSpec((1, Q_BLK, K), lambda bi, qi: (bi, qi, 0)),
        out_shape=jax.ShapeDtypeStruct((b, m, K), jnp.int32),
        scratch_shapes=[pltpu.VMEM((Q_BLK, n // LSUB, LSUB), jnp.float32)],
        compiler_params=pltpu.CompilerParams(
            dimension_semantics=("parallel", "parallel")),
    )(new_xyz, xyz_t)


# speedup vs baseline: 11.5222x; 11.5222x over previous
import jax
import jax.numpy as jnp
from jax.experimental import pallas as pl
from jax.experimental.pallas import tpu as pltpu

K = 32
Q_BLK = 128
LSUB = 128
BIGN = 1 << 30


def _ce(a, an, b, bn):
    sw = b < a
    return (jnp.where(sw, b, a), jnp.where(sw, bn, an),
            jnp.where(sw, a, b), jnp.where(sw, an, bn))


def _knn_kernel(p_ref, qt_ref, o_ref, d_ref):
    p = p_ref[0]
    qt = qt_ref[0]
    n = p.shape[0]
    s_rows = n // LSUB
    qn = jnp.sum(qt * qt, axis=0, keepdims=True)
    pn = jnp.sum(p * p, axis=1, keepdims=True)
    d = jax.lax.dot_general(
        p, qt, (((1,), (0,)), ((), ())),
        preferred_element_type=jnp.float32)
    d = (-2.0 * d + qn) + pn
    d_ref[...] = d.reshape(s_rows, LSUB, Q_BLK)

    inf = jnp.float32(jnp.inf)
    lane = jax.lax.broadcasted_iota(jnp.int32, (LSUB, Q_BLK), 0)
    k_sub = jax.lax.broadcasted_iota(jnp.int32, (K, Q_BLK), 0)

    def ins(s, carry):
        (v0, v1, v2, v3, v4, n0, n1, n2, n3, n4) = carry
        c = d_ref[s]
        cn = lane + s * LSUB
        lt = c < v4
        v4n = jnp.where(lt, c, v4)
        n4n = jnp.where(lt, cn, n4)
        v3, n3, v4, n4 = _ce(v3, n3, v4n, n4n)
        v2, n2, v3, n3 = _ce(v2, n2, v3, n3)
        v1, n1, v2, n2 = _ce(v1, n1, v2, n2)
        v0, n0, v1, n1 = _ce(v0, n0, v1, n1)
        return (v0, v1, v2, v3, v4, n0, n1, n2, n3, n4)

    finf = jnp.full((LSUB, Q_BLK), inf, jnp.float32)
    fbig = jnp.full((LSUB, Q_BLK), BIGN, jnp.int32)
    carry = (finf, finf, finf, finf, finf, fbig, fbig, fbig, fbig, fbig)
    (v0, v1, v2, v3, v4, n0, n1, n2, n3, n4) = jax.lax.fori_loop(
        0, s_rows, ins, carry)

    out = jnp.zeros((K, Q_BLK), jnp.int32)
    for k in range(K):
        mv = jnp.min(v0, axis=0, keepdims=True)
        sel = v0 == mv
        mn = jnp.min(jnp.where(sel, n0, BIGN), axis=0, keepdims=True)
        out = jnp.where(k_sub == k, mn, out)
        win = sel & (n0 == mn)
        v0 = jnp.where(win, v1, v0)
        n0 = jnp.where(win, n1, n0)
        v1 = jnp.where(win, v2, v1)
        n1 = jnp.where(win, n2, n1)
        v2 = jnp.where(win, v3, v2)
        n2 = jnp.where(win, n3, n2)
        v3 = jnp.where(win, v4, v3)
        n3 = jnp.where(win, n4, n3)
        v4 = jnp.where(win, inf, v4)
        n4 = jnp.where(win, BIGN, n4)
    o_ref[0] = out


def kernel(xyz, new_xyz):
    b, n, _ = xyz.shape
    m = new_xyz.shape[1]
    new_t = jnp.swapaxes(new_xyz, 1, 2)
    grid = (b, m // Q_BLK)
    out_t = pl.pallas_call(
        _knn_kernel,
        grid=grid,
        in_specs=[
            pl.BlockSpec((1, n, 3), lambda bi, qi: (bi, 0, 0)),
            pl.BlockSpec((1, 3, Q_BLK), lambda bi, qi: (bi, 0, qi)),
        ],
        out_specs=pl.BlockSpec((1, K, Q_BLK), lambda bi, qi: (bi, 0, qi)),
        out_shape=jax.ShapeDtypeStruct((b, K, m), jnp.int32),
        scratch_shapes=[pltpu.VMEM((n // LSUB, LSUB, Q_BLK), jnp.float32)],
        compiler_params=pltpu.CompilerParams(
            dimension_semantics=("parallel", "parallel")),
    )(xyz, new_t)
    return jnp.swapaxes(out_t, 1, 2)


# speedup vs baseline: 16.6491x; 1.4450x over previous
import jax
import jax.numpy as jnp
from jax.experimental import pallas as pl
from jax.experimental.pallas import tpu as pltpu

K = 32
Q_BLK = 128
LSUB = 128
BIGN = 1 << 30


def _ce(a, an, b, bn):
    sw = b < a
    return (jnp.where(sw, b, a), jnp.where(sw, bn, an),
            jnp.where(sw, a, b), jnp.where(sw, an, bn))


def _knn_kernel(p_ref, qt_ref, o_ref, d_ref):
    p = p_ref[0]
    qt = qt_ref[0]
    n = p.shape[0]
    s_rows = n // LSUB
    qn = jnp.sum(qt * qt, axis=0, keepdims=True)
    pn = jax.lax.dot_general(
        p * p, jnp.ones((3, 1), jnp.float32), (((1,), (0,)), ((), ())),
        preferred_element_type=jnp.float32)
    d = jax.lax.dot_general(
        p, qt, (((1,), (0,)), ((), ())),
        preferred_element_type=jnp.float32)
    d = (-2.0 * d + qn) + pn
    d_ref[...] = d.reshape(s_rows, LSUB, Q_BLK)

    inf = jnp.float32(jnp.inf)
    lane = jax.lax.broadcasted_iota(jnp.int32, (LSUB, Q_BLK), 0)
    k_sub = jax.lax.broadcasted_iota(jnp.int32, (K, Q_BLK), 0)

    def ins(s, carry):
        (v0, v1, v2, v3, v4, n0, n1, n2, n3, n4) = carry
        c = d_ref[s]
        cn = lane + s * LSUB
        lt = c < v4
        v4n = jnp.where(lt, c, v4)
        n4n = jnp.where(lt, cn, n4)
        v3, n3, v4, n4 = _ce(v3, n3, v4n, n4n)
        v2, n2, v3, n3 = _ce(v2, n2, v3, n3)
        v1, n1, v2, n2 = _ce(v1, n1, v2, n2)
        v0, n0, v1, n1 = _ce(v0, n0, v1, n1)
        return (v0, v1, v2, v3, v4, n0, n1, n2, n3, n4)

    finf = jnp.full((LSUB, Q_BLK), inf, jnp.float32)
    fbig = jnp.full((LSUB, Q_BLK), BIGN, jnp.int32)
    carry = (finf, finf, finf, finf, finf, fbig, fbig, fbig, fbig, fbig)
    (v0, v1, v2, v3, v4, n0, n1, n2, n3, n4) = jax.lax.fori_loop(
        0, s_rows, ins, carry, unroll=8)

    out = jnp.zeros((K, Q_BLK), jnp.int32)
    for k in range(K):
        mv = jnp.min(v0, axis=0, keepdims=True)
        sel = v0 == mv
        mn = jnp.min(jnp.where(sel, n0, BIGN), axis=0, keepdims=True)
        out = jnp.where(k_sub == k, mn, out)
        win = sel & (n0 == mn)
        v0 = jnp.where(win, v1, v0)
        n0 = jnp.where(win, n1, n0)
        v1 = jnp.where(win, v2, v1)
        n1 = jnp.where(win, n2, n1)
        v2 = jnp.where(win, v3, v2)
        n2 = jnp.where(win, n3, n2)
        v3 = jnp.where(win, v4, v3)
        n3 = jnp.where(win, n4, n3)
        v4 = jnp.where(win, inf, v4)
        n4 = jnp.where(win, BIGN, n4)
    o_ref[0] = out


def kernel(xyz, new_xyz):
    b, n, _ = xyz.shape
    m = new_xyz.shape[1]
    new_t = jnp.swapaxes(new_xyz, 1, 2)
    grid = (b, m // Q_BLK)
    out_t = pl.pallas_call(
        _knn_kernel,
        grid=grid,
        in_specs=[
            pl.BlockSpec((1, n, 3), lambda bi, qi: (bi, 0, 0)),
            pl.BlockSpec((1, 3, Q_BLK), lambda bi, qi: (bi, 0, qi)),
        ],
        out_specs=pl.BlockSpec((1, K, Q_BLK), lambda bi, qi: (bi, 0, qi)),
        out_shape=jax.ShapeDtypeStruct((b, K, m), jnp.int32),
        scratch_shapes=[pltpu.VMEM((n // LSUB, LSUB, Q_BLK), jnp.float32)],
        compiler_params=pltpu.CompilerParams(
            dimension_semantics=("parallel", "parallel")),
    )(xyz, new_t)
    return jnp.swapaxes(out_t, 1, 2)


# speedup vs baseline: 16.8695x; 1.0132x over previous
import jax
import jax.numpy as jnp
from jax.experimental import pallas as pl
from jax.experimental.pallas import tpu as pltpu

K = 32
Q_BLK = 128
LSUB = 128
BIGN = 1 << 30


def _ce(a, an, b, bn):
    sw = b < a
    return (jnp.where(sw, b, a), jnp.where(sw, bn, an),
            jnp.where(sw, a, b), jnp.where(sw, an, bn))


def _knn_kernel(p_ref, qt_ref, o_ref, d_ref):
    p = p_ref[0]
    qt = qt_ref[0]
    n = p.shape[0]
    s_rows = n // LSUB
    qn = jnp.sum(qt * qt, axis=0, keepdims=True)
    pn = jnp.sum(p * p, axis=1, keepdims=True)
    d = jax.lax.dot_general(
        p, qt, (((1,), (0,)), ((), ())),
        preferred_element_type=jnp.float32)
    d = (-2.0 * d + qn) + pn
    d_ref[...] = d.reshape(s_rows, LSUB, Q_BLK)

    inf = jnp.float32(jnp.inf)
    lane = jax.lax.broadcasted_iota(jnp.int32, (LSUB, Q_BLK), 0)
    k_sub = jax.lax.broadcasted_iota(jnp.int32, (K, Q_BLK), 0)

    def ins(s, carry):
        (v0, v1, v2, v3, v4, n0, n1, n2, n3, n4) = carry
        c = d_ref[s]
        cn = lane + s * LSUB
        lt = c < v4
        v4n = jnp.where(lt, c, v4)
        n4n = jnp.where(lt, cn, n4)
        v3, n3, v4, n4 = _ce(v3, n3, v4n, n4n)
        v2, n2, v3, n3 = _ce(v2, n2, v3, n3)
        v1, n1, v2, n2 = _ce(v1, n1, v2, n2)
        v0, n0, v1, n1 = _ce(v0, n0, v1, n1)
        return (v0, v1, v2, v3, v4, n0, n1, n2, n3, n4)

    finf = jnp.full((LSUB, Q_BLK), inf, jnp.float32)
    fbig = jnp.full((LSUB, Q_BLK), BIGN, jnp.int32)
    carry = (finf, finf, finf, finf, finf, fbig, fbig, fbig, fbig, fbig)
    (v0, v1, v2, v3, v4, n0, n1, n2, n3, n4) = jax.lax.fori_loop(
        0, s_rows, ins, carry, unroll=8)

    out = jnp.zeros((K, Q_BLK), jnp.int32)
    for k in range(K):
        mv = jnp.min(v0, axis=0, keepdims=True)
        sel = v0 == mv
        mn = jnp.min(jnp.where(sel, n0, BIGN), axis=0, keepdims=True)
        out = jnp.where(k_sub == k, mn, out)
        win = sel & (n0 == mn)
        v0 = jnp.where(win, v1, v0)
        n0 = jnp.where(win, n1, n0)
        v1 = jnp.where(win, v2, v1)
        n1 = jnp.where(win, n2, n1)
        v2 = jnp.where(win, v3, v2)
        n2 = jnp.where(win, n3, n2)
        v3 = jnp.where(win, v4, v3)
        n3 = jnp.where(win, n4, n3)
        v4 = jnp.where(win, inf, v4)
        n4 = jnp.where(win, BIGN, n4)
    o_ref[0] = out


def kernel(xyz, new_xyz):
    b, n, _ = xyz.shape
    m = new_xyz.shape[1]
    new_t = jnp.swapaxes(new_xyz, 1, 2)
    grid = (b, m // Q_BLK)
    out_t = pl.pallas_call(
        _knn_kernel,
        grid=grid,
        in_specs=[
            pl.BlockSpec((1, n, 3), lambda bi, qi: (bi, 0, 0)),
            pl.BlockSpec((1, 3, Q_BLK), lambda bi, qi: (bi, 0, qi)),
        ],
        out_specs=pl.BlockSpec((1, K, Q_BLK), lambda bi, qi: (bi, 0, qi)),
        out_shape=jax.ShapeDtypeStruct((b, K, m), jnp.int32),
        scratch_shapes=[pltpu.VMEM((n // LSUB, LSUB, Q_BLK), jnp.float32)],
        compiler_params=pltpu.CompilerParams(
            dimension_semantics=("parallel", "parallel")),
    )(xyz, new_t)
    return jnp.swapaxes(out_t, 1, 2)


# speedup vs baseline: 17.4042x; 1.0317x over previous
import jax
import jax.numpy as jnp
from jax.experimental import pallas as pl
from jax.experimental.pallas import tpu as pltpu

K = 32
Q_BLK = 128
LSUB = 128
BIGN = 1 << 30


def _ce(a, an, b, bn):
    sw = b < a
    return (jnp.where(sw, b, a), jnp.where(sw, bn, an),
            jnp.where(sw, a, b), jnp.where(sw, an, bn))


def _knn_kernel(p_ref, qt_ref, o_ref, d_ref):
    p = p_ref[0]
    qt = qt_ref[0]
    n = p.shape[0]
    s_rows = n // LSUB
    qn = jnp.sum(qt * qt, axis=0, keepdims=True)
    pn = jnp.sum(p * p, axis=1, keepdims=True)
    d = jax.lax.dot_general(
        p, qt, (((1,), (0,)), ((), ())),
        preferred_element_type=jnp.float32)
    d = (-2.0 * d + qn) + pn
    d_ref[...] = d.reshape(s_rows, LSUB, Q_BLK)

    inf = jnp.float32(jnp.inf)
    lane = jax.lax.broadcasted_iota(jnp.int32, (LSUB, Q_BLK), 0)
    k_sub = jax.lax.broadcasted_iota(jnp.int32, (K, Q_BLK), 0)

    def ins(s, carry):
        (v0, v1, v2, v3, v4, n0, n1, n2, n3, n4) = carry
        c = d_ref[s]
        cn = lane + s * LSUB
        lt = c < v4
        v4n = jnp.where(lt, c, v4)
        n4n = jnp.where(lt, cn, n4)
        v3, n3, v4, n4 = _ce(v3, n3, v4n, n4n)
        v2, n2, v3, n3 = _ce(v2, n2, v3, n3)
        v1, n1, v2, n2 = _ce(v1, n1, v2, n2)
        v0, n0, v1, n1 = _ce(v0, n0, v1, n1)
        return (v0, v1, v2, v3, v4, n0, n1, n2, n3, n4)

    finf = jnp.full((LSUB, Q_BLK), inf, jnp.float32)
    fbig = jnp.full((LSUB, Q_BLK), BIGN, jnp.int32)
    carry = (finf, finf, finf, finf, finf, fbig, fbig, fbig, fbig, fbig)
    (v0, v1, v2, v3, v4, n0, n1, n2, n3, n4) = jax.lax.fori_loop(
        0, s_rows, ins, carry, unroll=16)

    out = jnp.zeros((K, Q_BLK), jnp.int32)
    for k in range(K):
        mv = jnp.min(v0, axis=0, keepdims=True)
        sel = v0 == mv
        mn = jnp.min(jnp.where(sel, n0, BIGN), axis=0, keepdims=True)
        out = jnp.where(k_sub == k, mn, out)
        win = sel & (n0 == mn)
        v0 = jnp.where(win, v1, v0)
        n0 = jnp.where(win, n1, n0)
        v1 = jnp.where(win, v2, v1)
        n1 = jnp.where(win, n2, n1)
        v2 = jnp.where(win, v3, v2)
        n2 = jnp.where(win, n3, n2)
        v3 = jnp.where(win, v4, v3)
        n3 = jnp.where(win, n4, n3)
        v4 = jnp.where(win, inf, v4)
        n4 = jnp.where(win, BIGN, n4)
    o_ref[0] = out


def kernel(xyz, new_xyz):
    b, n, _ = xyz.shape
    m = new_xyz.shape[1]
    new_t = jnp.swapaxes(new_xyz, 1, 2)
    grid = (b, m // Q_BLK)
    out_t = pl.pallas_call(
        _knn_kernel,
        grid=grid,
        in_specs=[
            pl.BlockSpec((1, n, 3), lambda bi, qi: (bi, 0, 0)),
            pl.BlockSpec((1, 3, Q_BLK), lambda bi, qi: (bi, 0, qi)),
        ],
        out_specs=pl.BlockSpec((1, K, Q_BLK), lambda bi, qi: (bi, 0, qi)),
        out_shape=jax.ShapeDtypeStruct((b, K, m), jnp.int32),
        scratch_shapes=[pltpu.VMEM((n // LSUB, LSUB, Q_BLK), jnp.float32)],
        compiler_params=pltpu.CompilerParams(
            dimension_semantics=("parallel", "parallel")),
    )(xyz, new_t)
    return jnp.swapaxes(out_t, 1, 2)


# speedup vs baseline: 21.1610x; 1.2159x over previous
import jax
import jax.numpy as jnp
from jax.experimental import pallas as pl
from jax.experimental.pallas import tpu as pltpu

K = 32
Q_BLK = 128
LSUB = 128
BIGN = 1 << 30


def _ce(a, an, b, bn):
    sw = b < a
    return (jnp.where(sw, b, a), jnp.where(sw, bn, an),
            jnp.where(sw, a, b), jnp.where(sw, an, bn))


def _knn_kernel(p_ref, qt_ref, o_ref, d_ref):
    p = p_ref[0]
    qt = qt_ref[0]
    n = p.shape[0]
    s_rows = n // LSUB
    qn = jnp.sum(qt * qt, axis=0, keepdims=True)
    pn = jnp.sum(p * p, axis=1, keepdims=True)
    d = jax.lax.dot_general(
        p, qt, (((1,), (0,)), ((), ())),
        preferred_element_type=jnp.float32)
    d = (-2.0 * d + qn) + pn
    d_ref[...] = d.reshape(s_rows, LSUB, Q_BLK)

    inf = jnp.float32(jnp.inf)
    lane = jax.lax.broadcasted_iota(jnp.int32, (LSUB, Q_BLK), 0)
    k_sub = jax.lax.broadcasted_iota(jnp.int32, (K, Q_BLK), 0)

    def ins(s, carry):
        (v0, v1, v2, v3, n0, n1, n2, n3) = carry
        c = d_ref[s]
        cn = lane + s * LSUB
        lt = c < v3
        v3n = jnp.where(lt, c, v3)
        n3n = jnp.where(lt, cn, n3)
        v2, n2, v3, n3 = _ce(v2, n2, v3n, n3n)
        v1, n1, v2, n2 = _ce(v1, n1, v2, n2)
        v0, n0, v1, n1 = _ce(v0, n0, v1, n1)
        return (v0, v1, v2, v3, n0, n1, n2, n3)

    finf = jnp.full((LSUB, Q_BLK), inf, jnp.float32)
    fbig = jnp.full((LSUB, Q_BLK), BIGN, jnp.int32)
    carry = (finf, finf, finf, finf, fbig, fbig, fbig, fbig)
    (v0, v1, v2, v3, n0, n1, n2, n3) = jax.lax.fori_loop(
        0, s_rows, ins, carry, unroll=16)

    out = jnp.zeros((K, Q_BLK), jnp.int32)
    for k in range(K):
        mv = jnp.min(v0, axis=0, keepdims=True)
        sel = v0 == mv
        mn = jnp.min(jnp.where(sel, n0, BIGN), axis=0, keepdims=True)
        out = jnp.where(k_sub == k, mn, out)
        win = sel & (n0 == mn)
        v0 = jnp.where(win, v1, v0)
        n0 = jnp.where(win, n1, n0)
        v1 = jnp.where(win, v2, v1)
        n1 = jnp.where(win, n2, n1)
        v2 = jnp.where(win, v3, v2)
        n2 = jnp.where(win, n3, n2)
        v3 = jnp.where(win, inf, v3)
        n3 = jnp.where(win, BIGN, n3)
    o_ref[0] = out


def kernel(xyz, new_xyz):
    b, n, _ = xyz.shape
    m = new_xyz.shape[1]
    new_t = jnp.swapaxes(new_xyz, 1, 2)
    grid = (b, m // Q_BLK)
    out_t = pl.pallas_call(
        _knn_kernel,
        grid=grid,
        in_specs=[
            pl.BlockSpec((1, n, 3), lambda bi, qi: (bi, 0, 0)),
            pl.BlockSpec((1, 3, Q_BLK), lambda bi, qi: (bi, 0, qi)),
        ],
        out_specs=pl.BlockSpec((1, K, Q_BLK), lambda bi, qi: (bi, 0, qi)),
        out_shape=jax.ShapeDtypeStruct((b, K, m), jnp.int32),
        scratch_shapes=[pltpu.VMEM((n // LSUB, LSUB, Q_BLK), jnp.float32)],
        compiler_params=pltpu.CompilerParams(
            dimension_semantics=("parallel", "parallel")),
    )(xyz, new_t)
    return jnp.swapaxes(out_t, 1, 2)
